# baseline (speedup 1.0000x reference)
import functools

import jax
import jax.numpy as jnp
from jax import lax
from jax.experimental import pallas as pl
from jax.experimental.pallas import tpu as pltpu
from jax.experimental.pallas import tpu_sc as plsc

B = 16384
D_FRAME = 128
D_VIS = 64
D_EMB = 64
N_ACTIONS = 18

_NC = 2
_NS = 16
_NW = _NC * _NS
_BPW = B // _NW


@functools.cache
def _make_sc_gather():
    @functools.partial(
        pl.kernel,
        mesh=plsc.VectorSubcoreMesh(core_axis_name="c", subcore_axis_name="s"),
        out_type=jax.ShapeDtypeStruct((B, D_EMB), jnp.float32),
        scratch_types=[
            pltpu.VMEM((_BPW,), jnp.int32),
            pltpu.VMEM((_BPW, D_EMB), jnp.float32),
            pltpu.SemaphoreType.DMA,
        ],
        compiler_params=pltpu.CompilerParams(use_tc_tiling_on_sc=False),
    )
    def _sc_gather(idx_hbm, table_hbm, out_hbm, idx_v, rows_v, sem):
        wid = lax.axis_index("s") * _NC + lax.axis_index("c")
        base = wid * _BPW
        pltpu.sync_copy(idx_hbm.at[pl.ds(base, _BPW)], idx_v)
        pltpu.async_copy(table_hbm.at[idx_v], rows_v, sem).wait()
        pltpu.sync_copy(rows_v, out_hbm.at[pl.ds(base, _BPW)])

    return _sc_gather


def _dense_body(frames_ref, emb_ref, wvis_ref, bvis_ref, wpol_ref, bpol_ref,
                out_ref):
    vis = jnp.dot(frames_ref[...], wvis_ref[...],
                  preferred_element_type=jnp.float32) + bvis_ref[...]
    wp = wpol_ref[...]
    out_ref[...] = (
        jnp.dot(vis, wp[:D_VIS, :], preferred_element_type=jnp.float32)
        + jnp.dot(emb_ref[...], wp[D_VIS:, :],
                  preferred_element_type=jnp.float32)
        + bpol_ref[...]
    )


_BLK = 2048


def _dense(frames, emb, W_vis, b_vis2, W_pol, b_pol2):
    return pl.pallas_call(
        _dense_body,
        grid=(B // _BLK,),
        in_specs=[
            pl.BlockSpec((_BLK, D_FRAME), lambda i: (i, 0)),
            pl.BlockSpec((_BLK, D_EMB), lambda i: (i, 0)),
            pl.BlockSpec((D_FRAME, D_VIS), lambda i: (0, 0)),
            pl.BlockSpec((1, D_VIS), lambda i: (0, 0)),
            pl.BlockSpec((D_FRAME, N_ACTIONS), lambda i: (0, 0)),
            pl.BlockSpec((1, N_ACTIONS), lambda i: (0, 0)),
        ],
        out_specs=pl.BlockSpec((_BLK, N_ACTIONS), lambda i: (i, 0)),
        out_shape=jax.ShapeDtypeStruct((B, N_ACTIONS), jnp.float32),
    )(frames, emb, W_vis, b_vis2, W_pol, b_pol2)


def kernel(frames, object_index, W_vis, b_vis, emb_table, W_pol, b_pol):
    idx = object_index.astype(jnp.int32)
    emb = _make_sc_gather()(idx, emb_table)
    return _dense(frames, emb, W_vis, b_vis.reshape(1, D_VIS), W_pol,
                  b_pol.reshape(1, N_ACTIONS))
